# Initial kernel scaffold; baseline (speedup 1.0000x reference)
import jax
import jax.numpy as jnp
from jax.experimental import pallas as pl


def kernel(x, w_stem, b_stem, w1x1, b1x1, Wh, WwT, B_t1, bias4_t1, B_t2, bias4_t2, B_t3, bias4_t3):
    raise NotImplementedError("write your pallas kernel here")


# speedup vs baseline: 1.8716x; 1.8716x over previous
import functools

import jax
import jax.numpy as jnp
from jax.experimental import pallas as pl
from jax.experimental.pallas import tpu as pltpu


def _stem_kernel(a_ref, b_ref, bias_ref, o_ref):
    r = jnp.dot(a_ref[...], b_ref[...], preferred_element_type=jnp.float32)
    r = r + bias_ref[...]
    r = r * jax.nn.sigmoid(r)
    o_ref[...] = r.astype(o_ref.dtype)


def _stem_matmul(patches, w_stem, b_stem):
    M, K = patches.shape
    N = w_stem.shape[1]
    b_stem = b_stem.reshape(1, N)
    tm = min(512, M)
    grid = (M // tm,)
    return pl.pallas_call(
        _stem_kernel,
        out_shape=jax.ShapeDtypeStruct((M, N), jnp.bfloat16),
        grid=grid,
        in_specs=[
            pl.BlockSpec((tm, K), lambda m: (m, 0)),
            pl.BlockSpec((K, N), lambda m: (0, 0)),
            pl.BlockSpec((1, N), lambda m: (0, 0)),
        ],
        out_specs=pl.BlockSpec((tm, N), lambda m: (m, 0)),
        compiler_params=pltpu.CompilerParams(
            dimension_semantics=("parallel",),
            vmem_limit_bytes=56 * 1024 * 1024),
        cost_estimate=pl.CostEstimate(
            flops=2 * M * K * N, transcendentals=M * N,
            bytes_accessed=M * K * 2 + K * N * 2 + M * N * 2),
    )(patches, w_stem, b_stem)


def _deconv_kernel(aw_ref, b_ref, bias_ref, o_ref, *, H, W):
    C3 = aw_ref.shape[3]
    acc = jnp.dot(aw_ref[0, 0:H].reshape(H * W, C3), b_ref[0:C3],
                  preferred_element_type=jnp.float32)
    acc += jnp.dot(aw_ref[0, 1:H + 1].reshape(H * W, C3), b_ref[C3:2 * C3],
                   preferred_element_type=jnp.float32)
    acc += jnp.dot(aw_ref[0, 2:H + 2].reshape(H * W, C3), b_ref[2 * C3:3 * C3],
                   preferred_element_type=jnp.float32)
    r = jnp.maximum(acc + bias_ref[...], 0.0)
    o_ref[0] = r.astype(o_ref.dtype)


def _deconv_layer(aw, B9, bias4, H, W, n_split):
    N = aw.shape[0]
    C3 = aw.shape[3]
    NC = B9.shape[1]
    tn = NC // n_split
    bias_row = bias4.reshape(1, NC)
    grid = (n_split, N)
    kern = functools.partial(_deconv_kernel, H=H, W=W)
    flops = 2 * N * H * W * 3 * C3 * NC
    bytes_accessed = (N * (H + 2) * W * C3 * 2 * n_split + 3 * C3 * NC * 2
                      + N * H * W * NC * 2)
    return pl.pallas_call(
        kern,
        out_shape=jax.ShapeDtypeStruct((N, H * W, NC), jnp.bfloat16),
        grid=grid,
        in_specs=[
            pl.BlockSpec((1, H + 2, W, C3), lambda n, i: (i, 0, 0, 0)),
            pl.BlockSpec((3 * C3, tn), lambda n, i: (0, n)),
            pl.BlockSpec((1, tn), lambda n, i: (0, n)),
        ],
        out_specs=pl.BlockSpec((1, H * W, tn), lambda n, i: (i, 0, n)),
        compiler_params=pltpu.CompilerParams(
            dimension_semantics=("parallel", "arbitrary"),
            vmem_limit_bytes=56 * 1024 * 1024),
        cost_estimate=pl.CostEstimate(
            flops=flops, transcendentals=0,
            bytes_accessed=int(bytes_accessed)),
    )(aw, B9, bias_row)


def _deconv_head_kernel(aw_ref, b_ref, bias_ref, w14_ref, o_ref, *, H, W):
    C3 = aw_ref.shape[3]
    acc = jnp.dot(aw_ref[0, 0:H].reshape(H * W, C3), b_ref[0:C3],
                  preferred_element_type=jnp.float32)
    acc += jnp.dot(aw_ref[0, 1:H + 1].reshape(H * W, C3), b_ref[C3:2 * C3],
                   preferred_element_type=jnp.float32)
    acc += jnp.dot(aw_ref[0, 2:H + 2].reshape(H * W, C3), b_ref[2 * C3:3 * C3],
                   preferred_element_type=jnp.float32)
    r = jnp.maximum(acc + bias_ref[...], 0.0).astype(jnp.bfloat16)
    g = jnp.dot(r, w14_ref[...], preferred_element_type=jnp.float32)
    o_ref[0] = g.astype(o_ref.dtype)


def _deconv_head_layer(aw, B9, bias4, w14, H, W):
    N = aw.shape[0]
    C3 = aw.shape[3]
    NC = B9.shape[1]
    GC = w14.shape[1]
    bias_row = bias4.reshape(1, NC)
    kern = functools.partial(_deconv_head_kernel, H=H, W=W)
    flops = 2 * N * H * W * (3 * C3 * NC + NC * GC)
    bytes_accessed = (N * (H + 2) * W * C3 * 2 + 3 * C3 * NC * 2
                      + N * H * W * GC * 2)
    return pl.pallas_call(
        kern,
        out_shape=jax.ShapeDtypeStruct((N, H * W, GC), jnp.bfloat16),
        grid=(N,),
        in_specs=[
            pl.BlockSpec((1, H + 2, W, C3), lambda i: (i, 0, 0, 0)),
            pl.BlockSpec((3 * C3, NC), lambda i: (0, 0)),
            pl.BlockSpec((1, NC), lambda i: (0, 0)),
            pl.BlockSpec((NC, GC), lambda i: (0, 0)),
        ],
        out_specs=pl.BlockSpec((1, H * W, GC), lambda i: (i, 0, 0)),
        compiler_params=pltpu.CompilerParams(
            dimension_semantics=("parallel",),
            vmem_limit_bytes=56 * 1024 * 1024),
        cost_estimate=pl.CostEstimate(
            flops=flops, transcendentals=0,
            bytes_accessed=int(bytes_accessed)),
    )(aw, B9, bias_row, w14)


def _resize_kernel(g_ref, wh_ref, wwT_ref, bias_ref, o_ref):
    c = pl.program_id(1)
    t = jnp.dot(wh_ref[...], g_ref[0, 0], preferred_element_type=jnp.float32)
    y = jnp.dot(t.astype(jnp.bfloat16), wwT_ref[...],
                preferred_element_type=jnp.float32)
    o_ref[0, 0] = y + bias_ref[c]


def _head_resize(g_nchw, wh, wwT, bias):
    N, C, h, w = g_nchw.shape
    OH = wh.shape[0]
    OW = wwT.shape[1]
    flops = 2 * N * C * (OH * h * w + OH * w * OW)
    bytes_accessed = (N * C * h * w * 2 + OH * h * 2 + w * OW * 2
                      + N * C * OH * OW * 4)
    return pl.pallas_call(
        _resize_kernel,
        out_shape=jax.ShapeDtypeStruct((N, C, OH, OW), jnp.float32),
        grid=(N, C),
        in_specs=[
            pl.BlockSpec((1, 1, h, w), lambda n, c: (n, c, 0, 0)),
            pl.BlockSpec((OH, h), lambda n, c: (0, 0)),
            pl.BlockSpec((w, OW), lambda n, c: (0, 0)),
            pl.BlockSpec(memory_space=pltpu.MemorySpace.SMEM),
        ],
        out_specs=pl.BlockSpec((1, 1, OH, OW), lambda n, c: (n, c, 0, 0)),
        compiler_params=pltpu.CompilerParams(
            dimension_semantics=("parallel", "parallel"),
            vmem_limit_bytes=56 * 1024 * 1024),
        cost_estimate=pl.CostEstimate(
            flops=flops, transcendentals=0,
            bytes_accessed=int(bytes_accessed)),
    )(g_nchw, wh, wwT, bias.astype(jnp.float32))


def _width_cat(x_nhwc):
    xp = jnp.pad(x_nhwc, ((0, 0), (1, 1), (1, 1), (0, 0)))
    W = x_nhwc.shape[2]
    return jnp.concatenate([xp[:, :, b:b + W, :] for b in range(3)], axis=-1)


def _depth_to_space(y, H, W, C):
    N = y.shape[0]
    return (y.reshape(N, H, W, 2, 2, C)
             .transpose(0, 1, 3, 2, 4, 5)
             .reshape(N, 2 * H, 2 * W, C))


def kernel(x, w_stem, b_stem, w1x1, b1x1, Wh, WwT,
           B_t1, bias4_t1, B_t2, bias4_t2, B_t3, bias4_t3):
    N, C, H, W = x.shape
    P = 32
    hp, wp = H // P, W // P

    patches = (x.astype(jnp.bfloat16)
               .reshape(N, C, hp, P, wp, P)
               .transpose(0, 2, 4, 1, 3, 5)
               .reshape(N * hp * wp, C * P * P))
    f0 = _stem_matmul(patches, w_stem, b_stem).reshape(N, hp, wp, -1)

    aw1 = _width_cat(f0)
    y1 = _deconv_layer(aw1, B_t1, bias4_t1, hp, wp, n_split=4)
    f1 = _depth_to_space(y1, hp, wp, B_t1.shape[1] // 4)

    aw2 = _width_cat(f1)
    y2 = _deconv_layer(aw2, B_t2, bias4_t2, 2 * hp, 2 * wp, n_split=1)
    f2 = _depth_to_space(y2, 2 * hp, 2 * wp, B_t2.shape[1] // 4)

    aw3 = _width_cat(f2)
    nclass = w1x1.shape[1]
    cout3 = B_t3.shape[1] // 4
    w14 = jnp.zeros((4 * cout3, 4 * nclass), jnp.bfloat16)
    for ph in range(4):
        w14 = w14.at[ph * cout3:(ph + 1) * cout3,
                     ph * nclass:(ph + 1) * nclass].set(w1x1)
    y3 = _deconv_head_layer(aw3, B_t3, bias4_t3, w14, 4 * hp, 4 * wp)

    h3, w3 = 8 * hp, 8 * wp
    g = (y3.reshape(N, 4 * hp, 4 * wp, 2, 2, nclass)
           .transpose(0, 5, 1, 3, 2, 4)
           .reshape(N, nclass, h3, w3))
    return _head_resize(g, Wh, WwT, b1x1)
